# baseline (speedup 1.0000x reference)
import functools

import jax
import jax.numpy as jnp
from jax import lax
from jax.experimental import pallas as pl
from jax.experimental.pallas import tpu as pltpu
from jax.experimental.pallas import tpu_sc as plsc

N_NODES = 10000
D = 128
E = 320000
L = 16
W = D + L
NC = 2
NS = 16
S_PAD = 10240
S_WIN = 1024
NWIN = S_PAD // S_WIN
NPASS = NWIN // NC
A_CORE0 = 3
S_ACC = S_WIN + L
CHUNK = 128
E_PER_TILE = 20480
NCHUNK = E_PER_TILE // CHUNK
GRP = 8
NGRP = NCHUNK // GRP
E_PAD = NS * E_PER_TILE
ZROWS = S_ACC // NS
FROWS = S_WIN // NS
FBLK = 8

_MESH = plsc.VectorSubcoreMesh(core_axis_name="c", subcore_axis_name="s")


@functools.partial(
    pl.kernel,
    out_type=jax.ShapeDtypeStruct((S_PAD, D), jnp.float32),
    mesh=_MESH,
    scratch_types=[
        pltpu.VMEM((2, GRP, CHUNK), jnp.int32),
        pltpu.VMEM((CHUNK,), jnp.int32),
        pltpu.VMEM((CHUNK,), jnp.int32),
        pltpu.VMEM((CHUNK,), jnp.int32),
        pltpu.VMEM((CHUNK,), jnp.int32),
        pltpu.VMEM((CHUNK, W), jnp.float32),
        pltpu.VMEM((CHUNK, W), jnp.float32),
        pltpu.VMEM((FBLK, W), jnp.float32),
        pltpu.VMEM_SHARED((S_ACC, W), jnp.float32),
        pltpu.SemaphoreType.DMA,
        pltpu.SemaphoreType.DMA,
        pltpu.SemaphoreType.DMA,
        pltpu.SemaphoreType.DMA,
    ],
    compiler_params=pltpu.CompilerParams(
        needs_layout_passes=False, use_tc_tiling_on_sc=False),
)
def _agg(x_hbm, pkd_hbm, out_hbm,
         pbuf, idxrow0, idxrow1, segl0, segl1, buf0, buf1, sv,
         acc_sh, gsem, psem, ssem0, ssem1):
    idxrows = (idxrow0, idxrow1)
    segls = (segl0, segl1)
    ssems = (ssem0, ssem1)
    bufs = (buf0, buf1)
    cid = lax.axis_index("c")
    sid = lax.axis_index("s")
    meta = pkd_hbm.at[sid]

    zf = jnp.zeros((L,), jnp.float32)

    def _seg(vec):
        return jnp.right_shift(vec, 16)

    def _idx(vec):
        return jnp.bitwise_and(vec, jnp.int32(0xFFFF))

    def _pass(p, pcarry):
        wbase = (cid * A_CORE0 + p) * S_WIN

        def _zrow(r, carry):
            for k in range(W // L):
                buf0[r, pl.ds(k * L, L)] = zf
            return carry

        lax.fori_loop(0, ZROWS, _zrow, 0)
        zb = sid * ZROWS
        pltpu.sync_copy(buf0.at[pl.ds(0, ZROWS)],
                        acc_sh.at[pl.ds(zb, ZROWS)])
        plsc.subcore_barrier()

        def _relevant(vchunk_first, vchunk_last):
            first = jnp.min(_seg(vchunk_first))
            last = jnp.max(_seg(vchunk_last))
            return jnp.logical_and(last >= wbase, first < wbase + S_WIN)

        def _extract_idx(src_ref, c, b):
            for k in range(CHUNK // L):
                sl = pl.ds(k * L, L)
                idxrows[b][sl] = _idx(src_ref[c, sl])

        def _gather(b):
            return pltpu.make_async_copy(x_hbm.at[idxrows[b]], bufs[b], gsem)

        def _start_gather(b):
            pltpu.async_copy(x_hbm.at[idxrows[b]], bufs[b], gsem)

        def _wait_scatter(b):
            pltpu.make_async_copy(bufs[b], acc_sh.at[segls[b]],
                                  ssems[b]).wait()

        def _group(gg, gcarry):
            rprev, s0f, s1f = gcarry
            sflags = (s0f, s1f)
            for gb in range(2):
                g = 2 * gg + gb
                png = pbuf.at[1 - gb]

                @pl.when(g < NGRP - 1)
                def _():
                    pltpu.async_copy(meta.at[pl.ds((g + 1) * GRP, GRP)],
                                     png, psem)

                for c in range(GRP):
                    b = c % 2

                    if c == GRP - 1:
                        @pl.when(g < NGRP - 1)
                        def _():
                            pltpu.make_async_copy(
                                meta.at[pl.ds((g + 1) * GRP, GRP)], png,
                                psem).wait()

                    r = _relevant(pbuf[gb, c, pl.ds(0, L)],
                                  pbuf[gb, c, pl.ds(CHUNK - L, L)])
                    if c < GRP - 1:
                        rn = _relevant(pbuf[gb, c + 1, pl.ds(0, L)],
                                       pbuf[gb, c + 1, pl.ds(CHUNK - L, L)])
                        nref, nc_ = gb, c + 1
                    else:
                        rn = jnp.logical_and(
                            g < NGRP - 1,
                            _relevant(pbuf[1 - gb, 0, pl.ds(0, L)],
                                      pbuf[1 - gb, 0, pl.ds(CHUNK - L, L)]))
                        nref, nc_ = 1 - gb, 0

                    @pl.when(jnp.logical_and(r, jnp.logical_not(rprev)))
                    def _():
                        @pl.when(sflags[b])
                        def _():
                            _wait_scatter(b)

                        _extract_idx(pbuf.at[gb], c, b)
                        _start_gather(b)

                    @pl.when(r)
                    def _():
                        _gather(b).wait()

                        @pl.when(rn)
                        def _():
                            @pl.when(sflags[1 - b])
                            def _():
                                _wait_scatter(1 - b)

                            _extract_idx(pbuf.at[nref], nc_, 1 - b)
                            _start_gather(1 - b)

                        for k in range(CHUNK // L):
                            sl = pl.ds(k * L, L)
                            t = _seg(pbuf[gb, c, sl]) - wbase
                            ok = jnp.logical_and(t >= 0, t < S_WIN)
                            segls[b][sl] = jnp.where(ok, t, jnp.int32(S_WIN))
                        pltpu.async_copy(bufs[b], acc_sh.at[segls[b]],
                                         ssems[b], add=True)

                    new_sb = jnp.where(r, True, sflags[b])
                    new_snb = jnp.where(jnp.logical_and(r, rn),
                                        False, sflags[1 - b])
                    if b == 0:
                        sflags = (new_sb, new_snb)
                    else:
                        sflags = (new_snb, new_sb)
                    rprev = r
            return rprev, sflags[0], sflags[1]

        pltpu.sync_copy(meta.at[pl.ds(0, GRP)], pbuf.at[0])
        false = jnp.zeros((), jnp.bool_)
        _, s0f, s1f = lax.fori_loop(0, NGRP // 2, _group,
                                    (false, false, false))

        @pl.when(s0f)
        def _():
            _wait_scatter(0)

        @pl.when(s1f)
        def _():
            _wait_scatter(1)

        plsc.subcore_barrier()

        fb = sid * FROWS

        def _fin(blk, carry):
            rbase = fb + blk * FBLK
            pltpu.sync_copy(acc_sh.at[pl.ds(rbase, FBLK)], sv)
            for r in range(FBLK):
                rv = 1.0 / jnp.maximum(sv[r, pl.ds(D, L)], 1.0)
                for k in range(D // L):
                    sl = pl.ds(k * L, L)
                    sv[r, sl] = sv[r, sl] * rv
            pltpu.sync_copy(sv.at[:, pl.ds(0, D)],
                            out_hbm.at[pl.ds(wbase + rbase, FBLK)])
            return carry

        lax.fori_loop(0, FROWS // FBLK, _fin, 0)
        plsc.subcore_barrier()
        return pcarry

    npass = jnp.where(cid == 0, A_CORE0, NWIN - A_CORE0)
    lax.fori_loop(0, npass, _pass, 0)


def kernel(x, gather_index, segment_ids):
    gi = gather_index.astype(jnp.int32)
    si = segment_ids.astype(jnp.int32)
    pad = E_PAD - E
    gi = jnp.concatenate([gi, jnp.zeros((pad,), jnp.int32)])
    si = jnp.concatenate([si, jnp.full((pad,), N_NODES, jnp.int32)])
    pkd = jnp.bitwise_or(jnp.left_shift(si, 16), gi)
    pkd = pkd.reshape(NCHUNK, NS, CHUNK).transpose(1, 0, 2)
    xw = jnp.concatenate([x, jnp.ones((N_NODES, L), jnp.float32)], axis=1)
    out_pad = _agg(xw, pkd)
    return out_pad[:N_NODES]


# speedup vs baseline: 1.2134x; 1.2134x over previous
import functools

import jax
import jax.numpy as jnp
from jax import lax
from jax.experimental import pallas as pl
from jax.experimental.pallas import tpu as pltpu
from jax.experimental.pallas import tpu_sc as plsc

N_NODES = 10000
D = 128
E = 320000
L = 16
W = D + L
NC = 2
NS = 16
S_PAD = 10240
S_WIN = 1024
NWIN = S_PAD // S_WIN
NPASS = NWIN // NC
A_CORE0 = 7
S_ACC = S_WIN + L
CHUNK = 128
E_PER_TILE = 20480
NCHUNK = E_PER_TILE // CHUNK
GRP = 8
NGRP = NCHUNK // GRP
E_PAD = NS * E_PER_TILE
ZROWS = S_ACC // NS
FROWS = S_WIN // NS
FBLK = 8

_MESH = plsc.VectorSubcoreMesh(core_axis_name="c", subcore_axis_name="s")


@functools.partial(
    pl.kernel,
    out_type=jax.ShapeDtypeStruct((S_PAD, D), jnp.float32),
    mesh=_MESH,
    scratch_types=[
        pltpu.VMEM((2, GRP, CHUNK), jnp.int32),
        pltpu.VMEM((CHUNK,), jnp.int32),
        pltpu.VMEM((CHUNK,), jnp.int32),
        pltpu.VMEM((CHUNK,), jnp.int32),
        pltpu.VMEM((CHUNK,), jnp.int32),
        pltpu.VMEM((CHUNK, W), jnp.float32),
        pltpu.VMEM((CHUNK, W), jnp.float32),
        pltpu.VMEM((FBLK, W), jnp.float32),
        pltpu.VMEM_SHARED((S_ACC, W), jnp.float32),
        pltpu.SemaphoreType.DMA,
        pltpu.SemaphoreType.DMA,
        pltpu.SemaphoreType.DMA,
        pltpu.SemaphoreType.DMA,
    ],
    compiler_params=pltpu.CompilerParams(
        needs_layout_passes=False, use_tc_tiling_on_sc=False),
)
def _agg(x_hbm, pkd_hbm, out_hbm,
         pbuf, idxrow0, idxrow1, segl0, segl1, buf0, buf1, sv,
         acc_sh, gsem, psem, ssem0, ssem1):
    idxrows = (idxrow0, idxrow1)
    segls = (segl0, segl1)
    ssems = (ssem0, ssem1)
    bufs = (buf0, buf1)
    cid = lax.axis_index("c")
    sid = lax.axis_index("s")
    meta = pkd_hbm.at[sid]

    zf = jnp.zeros((L,), jnp.float32)

    def _seg(vec):
        return jnp.right_shift(vec, 16)

    def _idx(vec):
        return jnp.bitwise_and(vec, jnp.int32(0xFFFF))

    def _pass(p, pcarry):
        wbase = (cid * A_CORE0 + p) * S_WIN

        def _zrow(r, carry):
            for k in range(W // L):
                buf0[r, pl.ds(k * L, L)] = zf
            return carry

        lax.fori_loop(0, ZROWS, _zrow, 0)
        zb = sid * ZROWS
        pltpu.sync_copy(buf0.at[pl.ds(0, ZROWS)],
                        acc_sh.at[pl.ds(zb, ZROWS)])
        plsc.subcore_barrier()

        def _relevant(vchunk_first, vchunk_last):
            first = jnp.min(_seg(vchunk_first))
            last = jnp.max(_seg(vchunk_last))
            return jnp.logical_and(last >= wbase, first < wbase + S_WIN)

        def _extract_idx(src_ref, c, b):
            for k in range(CHUNK // L):
                sl = pl.ds(k * L, L)
                idxrows[b][sl] = _idx(src_ref[c, sl])

        def _gather(b):
            return pltpu.make_async_copy(x_hbm.at[idxrows[b]], bufs[b], gsem)

        def _start_gather(b):
            pltpu.async_copy(x_hbm.at[idxrows[b]], bufs[b], gsem)

        def _wait_scatter(b):
            pltpu.make_async_copy(bufs[b], acc_sh.at[segls[b]],
                                  ssems[b]).wait()

        def _group(gg, gcarry):
            rprev, s0f, s1f = gcarry
            sflags = (s0f, s1f)
            for gb in range(2):
                g = 2 * gg + gb
                png = pbuf.at[1 - gb]

                @pl.when(g < NGRP - 1)
                def _():
                    pltpu.async_copy(meta.at[pl.ds((g + 1) * GRP, GRP)],
                                     png, psem)

                for c in range(GRP):
                    b = c % 2

                    if c == GRP - 1:
                        @pl.when(g < NGRP - 1)
                        def _():
                            pltpu.make_async_copy(
                                meta.at[pl.ds((g + 1) * GRP, GRP)], png,
                                psem).wait()

                    r = _relevant(pbuf[gb, c, pl.ds(0, L)],
                                  pbuf[gb, c, pl.ds(CHUNK - L, L)])
                    if c < GRP - 1:
                        rn = _relevant(pbuf[gb, c + 1, pl.ds(0, L)],
                                       pbuf[gb, c + 1, pl.ds(CHUNK - L, L)])
                        nref, nc_ = gb, c + 1
                    else:
                        rn = jnp.logical_and(
                            g < NGRP - 1,
                            _relevant(pbuf[1 - gb, 0, pl.ds(0, L)],
                                      pbuf[1 - gb, 0, pl.ds(CHUNK - L, L)]))
                        nref, nc_ = 1 - gb, 0

                    @pl.when(jnp.logical_and(r, jnp.logical_not(rprev)))
                    def _():
                        @pl.when(sflags[b])
                        def _():
                            _wait_scatter(b)

                        _extract_idx(pbuf.at[gb], c, b)
                        _start_gather(b)

                    @pl.when(r)
                    def _():
                        _gather(b).wait()

                        @pl.when(rn)
                        def _():
                            @pl.when(sflags[1 - b])
                            def _():
                                _wait_scatter(1 - b)

                            _extract_idx(pbuf.at[nref], nc_, 1 - b)
                            _start_gather(1 - b)

                        for k in range(CHUNK // L):
                            sl = pl.ds(k * L, L)
                            t = _seg(pbuf[gb, c, sl]) - wbase
                            ok = jnp.logical_and(t >= 0, t < S_WIN)
                            segls[b][sl] = jnp.where(ok, t, jnp.int32(S_WIN))
                        pltpu.async_copy(bufs[b], acc_sh.at[segls[b]],
                                         ssems[b], add=True)

                    new_sb = jnp.where(r, True, sflags[b])
                    new_snb = jnp.where(jnp.logical_and(r, rn),
                                        False, sflags[1 - b])
                    if b == 0:
                        sflags = (new_sb, new_snb)
                    else:
                        sflags = (new_snb, new_sb)
                    rprev = r
            return rprev, sflags[0], sflags[1]

        pltpu.sync_copy(meta.at[pl.ds(0, GRP)], pbuf.at[0])
        false = jnp.zeros((), jnp.bool_)
        _, s0f, s1f = lax.fori_loop(0, NGRP // 2, _group,
                                    (false, false, false))

        @pl.when(s0f)
        def _():
            _wait_scatter(0)

        @pl.when(s1f)
        def _():
            _wait_scatter(1)

        plsc.subcore_barrier()

        fb = sid * FROWS

        def _fin(blk, carry):
            rbase = fb + blk * FBLK
            pltpu.sync_copy(acc_sh.at[pl.ds(rbase, FBLK)], sv)
            for r in range(FBLK):
                rv = 1.0 / jnp.maximum(sv[r, pl.ds(D, L)], 1.0)
                for k in range(D // L):
                    sl = pl.ds(k * L, L)
                    sv[r, sl] = sv[r, sl] * rv
            pltpu.sync_copy(sv.at[:, pl.ds(0, D)],
                            out_hbm.at[pl.ds(wbase + rbase, FBLK)])
            return carry

        lax.fori_loop(0, FROWS // FBLK, _fin, 0)
        plsc.subcore_barrier()
        return pcarry

    npass = jnp.where(cid == 0, A_CORE0, NWIN - A_CORE0)
    lax.fori_loop(0, npass, _pass, 0)


def kernel(x, gather_index, segment_ids):
    gi = gather_index.astype(jnp.int32)
    si = segment_ids.astype(jnp.int32)
    pad = E_PAD - E
    gi = jnp.concatenate([gi, jnp.zeros((pad,), jnp.int32)])
    si = jnp.concatenate([si, jnp.full((pad,), N_NODES, jnp.int32)])
    pkd = jnp.bitwise_or(jnp.left_shift(si, 16), gi)
    pkd = pkd.reshape(NCHUNK, NS, CHUNK).transpose(1, 0, 2)
    xw = jnp.concatenate([x, jnp.ones((N_NODES, L), jnp.float32)], axis=1)
    out_pad = _agg(xw, pkd)
    return out_pad[:N_NODES]


# speedup vs baseline: 1.3468x; 1.1099x over previous
import functools

import jax
import jax.numpy as jnp
from jax import lax
from jax.experimental import pallas as pl
from jax.experimental.pallas import tpu as pltpu
from jax.experimental.pallas import tpu_sc as plsc

N_NODES = 10000
D = 128
E = 320000
L = 16
NC = 2
NS = 16
S_PAD = 10240
S_WIN = 1024
NWIN = S_PAD // S_WIN
A_CORE0 = 7
S_ACC = S_WIN + L
CHUNK = 128
E_PER_TILE = 20480
NCHUNK = E_PER_TILE // CHUNK
SEG32 = 32
NCHUNK_PAD = 192
E_PAD = NS * E_PER_TILE
ZROWS = S_ACC // NS
FROWS = S_WIN // NS
FBLK = 8

_MESH = plsc.VectorSubcoreMesh(core_axis_name="c", subcore_axis_name="s")


@functools.partial(
    pl.kernel,
    out_type=jax.ShapeDtypeStruct((S_PAD, D), jnp.float32),
    mesh=_MESH,
    scratch_types=[
        pltpu.VMEM((SEG32, CHUNK), jnp.int32),
        pltpu.VMEM((CHUNK,), jnp.int32),
        pltpu.VMEM((CHUNK,), jnp.int32),
        pltpu.VMEM((CHUNK,), jnp.int32),
        pltpu.VMEM((CHUNK,), jnp.int32),
        pltpu.VMEM((CHUNK, D), jnp.float32),
        pltpu.VMEM((CHUNK, D), jnp.float32),
        pltpu.VMEM((CHUNK, L), jnp.float32),
        pltpu.VMEM((CHUNK, L), jnp.float32),
        pltpu.VMEM((FBLK, L), jnp.float32),
        pltpu.SMEM((NWIN, 2), jnp.int32),
        pltpu.VMEM_SHARED((S_ACC, D), jnp.float32),
        pltpu.VMEM_SHARED((S_ACC, L), jnp.float32),
        pltpu.SemaphoreType.DMA,
        pltpu.SemaphoreType.DMA,
        pltpu.SemaphoreType.DMA,
    ],
    compiler_params=pltpu.CompilerParams(
        needs_layout_passes=False, use_tc_tiling_on_sc=False),
)
def _agg(x_hbm, pkd_hbm, out_hbm,
         mbuf, idxrow0, idxrow1, segl0, segl1, buf0, buf1, ones_v, zero_v,
         cv, runs, sums_sh, cnt_sh, gsem, ssem0, ssem1):
    idxrows = (idxrow0, idxrow1)
    segls = (segl0, segl1)
    bufs = (buf0, buf1)
    ssems = (ssem0, ssem1)
    cid = lax.axis_index("c")
    sid = lax.axis_index("s")
    meta = pkd_hbm.at[sid]

    of = jnp.ones((L,), jnp.float32)
    zf = jnp.zeros((L,), jnp.float32)

    def _fill(r, carry):
        ones_v[r, :] = of
        zero_v[r, :] = zf
        return carry

    lax.fori_loop(0, CHUNK, _fill, 0)

    def _seg(vec):
        return jnp.right_shift(vec, 16)

    def _idx(vec):
        return jnp.bitwise_and(vec, jnp.int32(0xFFFF))

    def _winit(w, carry):
        runs[w, 0] = jnp.int32(NCHUNK)
        runs[w, 1] = jnp.int32(0)
        return carry

    lax.fori_loop(0, NWIN, _winit, 0)

    def _scan_seg(s, carry):
        pltpu.sync_copy(meta.at[pl.ds(s * SEG32, SEG32)], mbuf)
        for il in range(SEG32):
            j = s * SEG32 + il
            first = jnp.min(_seg(mbuf[il, pl.ds(0, L)]))
            last = jnp.max(_seg(mbuf[il, pl.ds(CHUNK - L, L)]))
            wf = jnp.right_shift(first, 10)
            wl = jnp.right_shift(last, 10)

            def _upd(w, c2):
                runs[w, 0] = jnp.minimum(runs[w, 0], j)
                runs[w, 1] = jnp.maximum(runs[w, 1], j + 1)
                return c2

            lax.fori_loop(wf, wl + 1, _upd, 0)
        return carry

    lax.fori_loop(0, NCHUNK // SEG32, _scan_seg, 0)

    def _pass(p, pcarry):
        w = cid * A_CORE0 + p
        wbase = w * S_WIN

        def _zrow(r, carry):
            for k in range(D // L):
                buf0[r, pl.ds(k * L, L)] = zf
            return carry

        lax.fori_loop(0, ZROWS, _zrow, 0)
        zb = sid * ZROWS
        pltpu.sync_copy(buf0.at[pl.ds(0, ZROWS)],
                        sums_sh.at[pl.ds(zb, ZROWS)])
        pltpu.sync_copy(zero_v.at[pl.ds(0, ZROWS)],
                        cnt_sh.at[pl.ds(zb, ZROWS)])
        plsc.subcore_barrier()

        lo = runs[w, 0]
        hi = runs[w, 1]
        n = jnp.maximum(hi - lo, 0)
        nseg = (n + SEG32 - 1) // SEG32

        def _extract(il, b):
            for k in range(CHUNK // L):
                sl = pl.ds(k * L, L)
                idxrows[b][sl] = _idx(mbuf[il, sl])

        def _start_gather(b):
            pltpu.async_copy(x_hbm.at[idxrows[b]], bufs[b], gsem)

        def _wait_gather(b):
            pltpu.make_async_copy(x_hbm.at[idxrows[b]], bufs[b], gsem).wait()

        def _wait_scatter(b):
            pltpu.make_async_copy(bufs[b], sums_sh.at[segls[b]],
                                  ssems[b]).wait()
            pltpu.make_async_copy(ones_v, cnt_sh.at[segls[b]],
                                  ssems[b]).wait()

        def _segment(s, carry):
            s0f, s1f = carry
            sflags = (s0f, s1f)
            base = lo + s * SEG32
            pltpu.sync_copy(meta.at[pl.ds(base, SEG32)], mbuf)

            @pl.when(s * SEG32 < n)
            def _():
                @pl.when(sflags[0])
                def _():
                    _wait_scatter(0)

                _extract(0, 0)
                _start_gather(0)

            new0 = jnp.where(s * SEG32 < n, False, sflags[0])
            sflags = (new0, sflags[1])

            for il in range(SEG32):
                i = s * SEG32 + il
                b = il % 2
                valid = i < n

                @pl.when(valid)
                def _():
                    _wait_gather(b)

                    if il < SEG32 - 1:
                        @pl.when(i + 1 < n)
                        def _():
                            @pl.when(sflags[1 - b])
                            def _():
                                _wait_scatter(1 - b)

                            _extract(il + 1, 1 - b)
                            _start_gather(1 - b)

                    for k in range(CHUNK // L):
                        sl = pl.ds(k * L, L)
                        t = _seg(mbuf[il, sl]) - wbase
                        ok = jnp.logical_and(t >= 0, t < S_WIN)
                        segls[b][sl] = jnp.where(ok, t, jnp.int32(S_WIN))
                    pltpu.async_copy(bufs[b], sums_sh.at[segls[b]],
                                     ssems[b], add=True)
                    pltpu.async_copy(ones_v, cnt_sh.at[segls[b]],
                                     ssems[b], add=True)

                nb = jnp.where(valid, True, sflags[b])
                if il < SEG32 - 1:
                    lookahead = jnp.logical_and(valid, i + 1 < n)
                else:
                    lookahead = jnp.zeros((), jnp.bool_)
                nnb = jnp.where(lookahead, False, sflags[1 - b])
                sflags = (nb, nnb) if b == 0 else (nnb, nb)
            return sflags

        false = jnp.zeros((), jnp.bool_)
        s0f, s1f = lax.fori_loop(0, nseg, _segment, (false, false))

        @pl.when(s0f)
        def _():
            _wait_scatter(0)

        @pl.when(s1f)
        def _():
            _wait_scatter(1)

        plsc.subcore_barrier()

        fb = sid * FROWS

        def _fin(blk, carry):
            rbase = fb + blk * FBLK
            pltpu.sync_copy(sums_sh.at[pl.ds(rbase, FBLK)],
                            buf0.at[pl.ds(0, FBLK)])
            pltpu.sync_copy(cnt_sh.at[pl.ds(rbase, FBLK)], cv)
            for r in range(FBLK):
                rv = 1.0 / jnp.maximum(cv[r, :], 1.0)
                for k in range(D // L):
                    sl = pl.ds(k * L, L)
                    buf0[r, sl] = buf0[r, sl] * rv
            pltpu.sync_copy(buf0.at[pl.ds(0, FBLK)],
                            out_hbm.at[pl.ds(wbase + rbase, FBLK)])
            return carry

        lax.fori_loop(0, FROWS // FBLK, _fin, 0)
        plsc.subcore_barrier()
        return pcarry

    npass = jnp.where(cid == 0, A_CORE0, NWIN - A_CORE0)
    lax.fori_loop(0, npass, _pass, 0)


def kernel(x, gather_index, segment_ids):
    gi = gather_index.astype(jnp.int32)
    si = segment_ids.astype(jnp.int32)
    pad = E_PAD - E
    gi = jnp.concatenate([gi, jnp.zeros((pad,), jnp.int32)])
    si = jnp.concatenate([si, jnp.full((pad,), N_NODES, jnp.int32)])
    pkd = jnp.bitwise_or(jnp.left_shift(si, 16), gi)
    pkd = pkd.reshape(NCHUNK, NS, CHUNK).transpose(1, 0, 2)
    pkd = jnp.concatenate(
        [pkd, jnp.zeros((NS, NCHUNK_PAD - NCHUNK, CHUNK), jnp.int32)], axis=1)
    out_pad = _agg(x, pkd)
    return out_pad[:N_NODES]
